# baseline (speedup 1.0000x reference)
import functools

import jax
import jax.numpy as jnp
from jax import lax
from jax.experimental import pallas as pl
from jax.experimental.pallas import tpu as pltpu
from jax.experimental.pallas import tpu_sc as plsc

B = 16384
D = 64
LANES = 16
PW = 2 * D
NC = 2
NS = 16
NW = NC * NS
N = 1000000

BPW = B // NW
CH = 128
NCH = BPW // CH
GPC = CH // LANES

RBLK = 16384
S_TC = 45 * RBLK
NSC = 262144
S2 = S_TC + NSC
TAILN = N - S2
SCCH = 128
NFULL = NSC // SCCH
GPW = NFULL // NW


def _tc_repack_body(ut_ref, it_ref, pc_ref):
    pc_ref[...] = jnp.concatenate(
        [jnp.transpose(ut_ref[...], (1, 0)),
         jnp.transpose(it_ref[...], (1, 0))], axis=1)


def _tc_repack_tail(ut_ref, it_ref, pc_ref):
    pc_ref[...] = jnp.concatenate(
        [jnp.transpose(ut_ref[...], (1, 0)),
         jnp.transpose(it_ref[...], (1, 0))], axis=1)


def _sc_repack_body(ut_hbm, it_hbm, pc2_hbm, ubT, ibT, stg, sem):
    wid = lax.axis_index("s") * NC + lax.axis_index("c")
    lane = lax.iota(jnp.int32, LANES)

    def extract(src_u, src_i, nids):
        for f in range(D):
            for g in range(nids // LANES):
                rows = g * LANES + lane
                u16 = src_u[f, pl.ds(g * LANES, LANES)]
                i16 = src_i[f, pl.ds(g * LANES, LANES)]
                plsc.store_scatter(stg, [rows, jnp.full((LANES,), f, jnp.int32)], u16)
                plsc.store_scatter(stg, [rows, jnp.full((LANES,), D + f, jnp.int32)], i16)

    def chunk_body(g, carry):
        c = wid + NW * g
        col0 = pl.multiple_of((S_TC // SCCH + c) * SCCH, SCCH)
        cu = pltpu.async_copy(ut_hbm.at[:, pl.ds(col0, SCCH)], ubT, sem)
        ci = pltpu.async_copy(it_hbm.at[:, pl.ds(col0, SCCH)], ibT, sem)
        cu.wait()
        ci.wait()
        extract(ubT, ibT, SCCH)
        pltpu.sync_copy(stg, pc2_hbm.at[pl.ds(c * SCCH, SCCH)])
        return carry

    lax.fori_loop(0, GPW, chunk_body, 0)


def _sc_gather_body(uid_hbm, iid_hbm, pc1_hbm, pc2_hbm, pc3_hbm, out_hbm,
                    uidv, iidv, ugxa, ugxb, ugxc, igxa, igxb, igxc,
                    uba, ubb, ubc, iba, ibb, ibc, outv, sem):
    wid = lax.axis_index("s") * NC + lax.axis_index("c")
    base = wid * BPW

    pltpu.sync_copy(uid_hbm.at[pl.ds(base, BPW)], uidv)
    pltpu.sync_copy(iid_hbm.at[pl.ds(base, BPW)], iidv)

    for g in range(BPW // LANES):
        r, c = g // (CH // LANES), (g % (CH // LANES)) * LANES
        u16 = uidv[pl.ds(g * LANES, LANES)]
        i16 = iidv[pl.ds(g * LANES, LANES)]
        ugxa[r, pl.ds(c, LANES)] = jnp.minimum(u16, S_TC - 1)
        ugxb[r, pl.ds(c, LANES)] = jnp.clip(u16 - S_TC, 0, NSC - 1)
        ugxc[r, pl.ds(c, LANES)] = jnp.clip(u16 - S2, 0, TAILN - 1)
        igxa[r, pl.ds(c, LANES)] = jnp.minimum(i16, S_TC - 1)
        igxb[r, pl.ds(c, LANES)] = jnp.clip(i16 - S_TC, 0, NSC - 1)
        igxc[r, pl.ds(c, LANES)] = jnp.clip(i16 - S2, 0, TAILN - 1)

    lane = lax.iota(jnp.int32, LANES)

    for ch in range(NCH):
        copies = [
            pltpu.async_copy(pc1_hbm.at[ugxa.at[ch]], uba, sem),
            pltpu.async_copy(pc2_hbm.at[ugxb.at[ch]], ubb, sem),
            pltpu.async_copy(pc3_hbm.at[ugxc.at[ch]], ubc, sem),
            pltpu.async_copy(pc1_hbm.at[igxa.at[ch]], iba, sem),
            pltpu.async_copy(pc2_hbm.at[igxb.at[ch]], ibb, sem),
            pltpu.async_copy(pc3_hbm.at[igxc.at[ch]], ibc, sem),
        ]
        for cp in copies:
            cp.wait()

        def group(g, carry):
            rows = g * LANES + lane
            u16 = uidv[pl.ds(ch * CH + g * LANES, LANES)]
            i16 = iidv[pl.ds(ch * CH + g * LANES, LANES)]
            acc = jnp.zeros((LANES,), jnp.float32)
            for d in range(D):
                ucol = jnp.full((LANES,), d, jnp.int32)
                icol = jnp.full((LANES,), D + d, jnp.int32)
                uu = jnp.where(
                    u16 < S_TC, plsc.load_gather(uba, [rows, ucol]),
                    jnp.where(u16 < S2, plsc.load_gather(ubb, [rows, ucol]),
                              plsc.load_gather(ubc, [rows, ucol])))
                vv = jnp.where(
                    i16 < S_TC, plsc.load_gather(iba, [rows, icol]),
                    jnp.where(i16 < S2, plsc.load_gather(ibb, [rows, icol]),
                              plsc.load_gather(ibc, [rows, icol])))
                acc = acc + uu * vv
            outv[pl.ds(ch * CH + g * LANES, LANES)] = acc
            return carry

        lax.fori_loop(0, GPC, group, 0)

    pltpu.sync_copy(outv, out_hbm.at[pl.ds(base, BPW)])


def kernel(user_ids, item_ids, user_table, item_table):
    utT = user_table.T
    itT = item_table.T
    mesh = plsc.VectorSubcoreMesh(core_axis_name="c", subcore_axis_name="s")

    pc2 = functools.partial(
        pl.kernel,
        mesh=mesh,
        compiler_params=pltpu.CompilerParams(needs_layout_passes=False),
        out_type=jax.ShapeDtypeStruct((NSC, PW), jnp.float32),
        scratch_types=[
            pltpu.VMEM((D, SCCH), jnp.float32),
            pltpu.VMEM((D, SCCH), jnp.float32),
            pltpu.VMEM((SCCH, PW), jnp.float32),
            pltpu.SemaphoreType.DMA,
        ],
    )(_sc_repack_body)(utT, itT)

    pc1 = pl.pallas_call(
        _tc_repack_body,
        grid=(S_TC // RBLK,),
        in_specs=[
            pl.BlockSpec((D, RBLK), lambda g: (0, g)),
            pl.BlockSpec((D, RBLK), lambda g: (0, g)),
        ],
        out_specs=pl.BlockSpec((RBLK, PW), lambda g: (g, 0)),
        out_shape=jax.ShapeDtypeStruct((S_TC, PW), jnp.float32),
    )(utT, itT)

    utT_tail = lax.slice(utT, (0, S2), (D, N))
    itT_tail = lax.slice(itT, (0, S2), (D, N))
    pc3 = pl.pallas_call(
        lambda u_ref, i_ref, o_ref: _tc_repack_tail(u_ref, i_ref, o_ref),
        grid=(1,),
        in_specs=[
            pl.BlockSpec((D, TAILN), lambda g: (0, 0)),
            pl.BlockSpec((D, TAILN), lambda g: (0, 0)),
        ],
        out_specs=pl.BlockSpec((TAILN, PW), lambda g: (0, 0)),
        out_shape=jax.ShapeDtypeStruct((TAILN, PW), jnp.float32),
    )(utT_tail, itT_tail)

    run = functools.partial(
        pl.kernel,
        mesh=mesh,
        compiler_params=pltpu.CompilerParams(needs_layout_passes=False),
        out_type=jax.ShapeDtypeStruct((B,), jnp.float32),
        scratch_types=[
            pltpu.VMEM((BPW,), jnp.int32),
            pltpu.VMEM((BPW,), jnp.int32),
            pltpu.VMEM((NCH, CH), jnp.int32),
            pltpu.VMEM((NCH, CH), jnp.int32),
            pltpu.VMEM((NCH, CH), jnp.int32),
            pltpu.VMEM((NCH, CH), jnp.int32),
            pltpu.VMEM((NCH, CH), jnp.int32),
            pltpu.VMEM((NCH, CH), jnp.int32),
            pltpu.VMEM((CH, PW), jnp.float32),
            pltpu.VMEM((CH, PW), jnp.float32),
            pltpu.VMEM((CH, PW), jnp.float32),
            pltpu.VMEM((CH, PW), jnp.float32),
            pltpu.VMEM((CH, PW), jnp.float32),
            pltpu.VMEM((CH, PW), jnp.float32),
            pltpu.VMEM((BPW,), jnp.float32),
            pltpu.SemaphoreType.DMA,
        ],
    )(_sc_gather_body)
    return run(user_ids.astype(jnp.int32), item_ids.astype(jnp.int32),
               pc1, pc2, pc3)


# speedup vs baseline: 4.5348x; 4.5348x over previous
import functools

import jax
import jax.numpy as jnp
from jax import lax
from jax.experimental import pallas as pl
from jax.experimental.pallas import tpu as pltpu
from jax.experimental.pallas import tpu_sc as plsc

B = 16384
D = 64
LANES = 16
PACK = 2
PW = PACK * D
NC = 2
NS = 16
NW = NC * NS
BPW = B // NW
CH = 128
NCH = BPW // CH
IDC = 128
GPC = CH // LANES

RBLK = 16384


def _repack_body(ut_ref, it_ref, pc_ref):
    pc_ref[...] = jnp.concatenate(
        [jnp.transpose(ut_ref[...], (1, 0)),
         jnp.transpose(it_ref[...], (1, 0))], axis=1)


def _sc_body(uid_hbm, iid_hbm, pc_hbm, out_hbm,
             uidv, iidv, ugidx, igidx,
             ubuf0, ibuf0, ubuf1, ibuf1, outv, sem0, sem1):
    wid = lax.axis_index("s") * NC + lax.axis_index("c")
    base = wid * BPW

    pltpu.sync_copy(uid_hbm.at[pl.ds(base, BPW)], uidv)
    pltpu.sync_copy(iid_hbm.at[pl.ds(base, BPW)], iidv)

    for g in range(BPW // LANES):
        r, c = g // (IDC // LANES), (g % (IDC // LANES)) * LANES
        ugidx[r, pl.ds(c, LANES)] = uidv[pl.ds(g * LANES, LANES)]
        igidx[r, pl.ds(c, LANES)] = iidv[pl.ds(g * LANES, LANES)]

    lane = lax.iota(jnp.int32, LANES)
    bufs = [(ubuf0, ibuf0, sem0), (ubuf1, ibuf1, sem1)]

    def fire(ch):
        ub, ib, sem = bufs[ch % 2]
        return (pltpu.async_copy(pc_hbm.at[ugidx.at[ch]], ub, sem),
                pltpu.async_copy(pc_hbm.at[igidx.at[ch]], ib, sem))

    inflight = fire(0)
    for ch in range(NCH):
        for cp in inflight:
            cp.wait()
        if ch + 1 < NCH:
            inflight = fire(ch + 1)
        ub, ib, _ = bufs[ch % 2]

        def group(g, carry):
            rows = g * LANES + lane
            acc = jnp.zeros((LANES,), jnp.float32)
            for d in range(D):
                ucol = jnp.full((LANES,), d, jnp.int32)
                icol = jnp.full((LANES,), D + d, jnp.int32)
                uu = plsc.load_gather(ub, [rows, ucol])
                vv = plsc.load_gather(ib, [rows, icol])
                acc = acc + uu * vv
            outv[pl.ds(ch * CH + g * LANES, LANES)] = acc
            return carry

        lax.fori_loop(0, GPC, group, 0)

    pltpu.sync_copy(outv, out_hbm.at[pl.ds(base, BPW)])


def kernel(user_ids, item_ids, user_table, item_table):
    n = user_table.shape[0]
    utT = user_table.T
    itT = item_table.T
    grid = pl.cdiv(n, RBLK)

    pc = pl.pallas_call(
        _repack_body,
        grid=(grid,),
        in_specs=[
            pl.BlockSpec((D, RBLK), lambda g: (0, g)),
            pl.BlockSpec((D, RBLK), lambda g: (0, g)),
        ],
        out_specs=pl.BlockSpec((RBLK, PW), lambda g: (g, 0)),
        out_shape=jax.ShapeDtypeStruct((n, PW), jnp.float32),
    )(utT, itT)

    mesh = plsc.VectorSubcoreMesh(core_axis_name="c", subcore_axis_name="s")
    run = functools.partial(
        pl.kernel,
        mesh=mesh,
        compiler_params=pltpu.CompilerParams(needs_layout_passes=False),
        out_type=jax.ShapeDtypeStruct((B,), jnp.float32),
        scratch_types=[
            pltpu.VMEM((BPW,), jnp.int32),
            pltpu.VMEM((BPW,), jnp.int32),
            pltpu.VMEM((BPW // IDC, IDC), jnp.int32),
            pltpu.VMEM((BPW // IDC, IDC), jnp.int32),
            pltpu.VMEM((CH, PW), jnp.float32),
            pltpu.VMEM((CH, PW), jnp.float32),
            pltpu.VMEM((CH, PW), jnp.float32),
            pltpu.VMEM((CH, PW), jnp.float32),
            pltpu.VMEM((BPW,), jnp.float32),
            pltpu.SemaphoreType.DMA,
            pltpu.SemaphoreType.DMA,
        ],
    )(_sc_body)
    return run(user_ids.astype(jnp.int32), item_ids.astype(jnp.int32), pc)
